# baseline (speedup 1.0000x reference)
import functools

import jax
import jax.numpy as jnp
from jax import lax
from jax.experimental import pallas as pl
from jax.experimental.pallas import tpu as pltpu
from jax.experimental.pallas import tpu_sc as plsc

N_NODES = 10000
N_EDGES = 320000
D_FEAT = 128
N_CORES = 2
N_SUBCORES = 16
NW = N_CORES * N_SUBCORES
EPT = N_EDGES // NW
CHUNK = 128
NCHUNK = -(-EPT // CHUNK)
EPT_PAD = NCHUNK * CHUNK
TAIL = EPT_PAD - EPT

_ROWS_PER_BLOCK = 2048
N_PAD = 10240


def _rowsum_body(x_ref, r_ref):
    r_ref[...] = jnp.sum(x_ref[...], axis=1)


def _rowsum(x):
    grid = N_PAD // _ROWS_PER_BLOCK
    return pl.pallas_call(
        _rowsum_body,
        grid=(grid,),
        in_specs=[pl.BlockSpec((_ROWS_PER_BLOCK, D_FEAT), lambda i: (i, 0))],
        out_specs=pl.BlockSpec((_ROWS_PER_BLOCK,), lambda i: (i,)),
        out_shape=jax.ShapeDtypeStruct((N_PAD,), jnp.float32),
    )(x)


def _sc_scatter(r, ei4, zeros):
    mesh = plsc.VectorSubcoreMesh(core_axis_name="c", subcore_axis_name="s")

    @functools.partial(
        pl.kernel,
        out_type=jax.ShapeDtypeStruct((N_CORES, N_NODES), jnp.float32),
        mesh=mesh,
        scratch_types=[
            pltpu.VMEM((NCHUNK, CHUNK), jnp.int32),
            pltpu.VMEM((NCHUNK, CHUNK), jnp.int32),
            pltpu.VMEM((NCHUNK, CHUNK), jnp.float32),
            pltpu.VMEM_SHARED((N_PAD,), jnp.float32),
            pltpu.VMEM_SHARED((N_NODES,), jnp.float32),
            pltpu.SemaphoreType.DMA,
            pltpu.SemaphoreType.DMA,
        ],
    )
    def scatter_kernel(r_hbm, ei_hbm, z_hbm, out_hbm,
                       src_v, dst_v, val_v, r_sh, acc_sh, gsem, ssem):
        cid = lax.axis_index("c")
        sid = lax.axis_index("s")
        wid = sid * N_CORES + cid

        @pl.when(sid == 0)
        def _():
            pltpu.sync_copy(r_hbm, r_sh)
            pltpu.sync_copy(z_hbm, acc_sh)

        pltpu.sync_copy(ei_hbm.at[0, wid], src_v)
        pltpu.sync_copy(ei_hbm.at[1, wid], dst_v)
        plsc.subcore_barrier()

        def fire_gather(j, carry):
            pltpu.async_copy(r_sh.at[src_v.at[j]], val_v.at[j], gsem)
            return carry

        lax.fori_loop(0, NCHUNK, fire_gather, 0)

        def drain_gather(j, carry):
            pltpu.make_async_copy(r_sh.at[src_v.at[j]], val_v.at[j],
                                  gsem).wait()
            return carry

        lax.fori_loop(0, NCHUNK, drain_gather, 0)

        for k in range(TAIL // 16):
            val_v[NCHUNK - 1, pl.ds(EPT % CHUNK + 16 * k, 16)] = (
                jnp.zeros((16,), jnp.float32))

        def fire_scatter(j, carry):
            pltpu.async_copy(val_v.at[j], acc_sh.at[dst_v.at[j]], ssem,
                             add=True)
            return carry

        lax.fori_loop(0, NCHUNK, fire_scatter, 0)

        def drain_scatter(j, carry):
            pltpu.make_async_copy(val_v.at[j], acc_sh.at[dst_v.at[j]],
                                  ssem).wait()
            return carry

        lax.fori_loop(0, NCHUNK, drain_scatter, 0)
        plsc.subcore_barrier()

        @pl.when(sid == 0)
        def _():
            pltpu.sync_copy(acc_sh, out_hbm.at[cid])

    return scatter_kernel(r, ei4, zeros)


def _final_body(p_ref, o_ref):
    s = p_ref[0, :] + p_ref[1, :]
    t = jnp.where(s > 0, s, 0.1 * s)
    h = 16.0 * t
    u = jnp.where(h > 0, h, 0.1 * h)
    o_ref[...] = (10.0 * u).astype(jnp.int32)


def _finalize(parts):
    return pl.pallas_call(
        _final_body,
        out_shape=jax.ShapeDtypeStruct((N_NODES,), jnp.int32),
    )(parts)


def kernel(x, edge_index):
    ei3 = edge_index.astype(jnp.int32).reshape(2, NW, EPT)
    ei3 = jnp.pad(ei3, ((0, 0), (0, 0), (0, TAIL)))
    ei4 = ei3.reshape(2, NW, NCHUNK, CHUNK)
    zeros = jnp.zeros((N_NODES,), jnp.float32)
    r = _rowsum(x)
    parts = _sc_scatter(r, ei4, zeros)
    return _finalize(parts)


# speedup vs baseline: 1.1050x; 1.1050x over previous
import functools

import jax
import jax.numpy as jnp
from jax import lax
from jax.experimental import pallas as pl
from jax.experimental.pallas import tpu as pltpu
from jax.experimental.pallas import tpu_sc as plsc

N_NODES = 10000
N_EDGES = 320000
D_FEAT = 128
N_CORES = 2
N_SUBCORES = 16
NW = N_CORES * N_SUBCORES
EPT = N_EDGES // NW
CHUNK = 80
NCHUNK = EPT // CHUNK

_ROWS_PER_BLOCK = 2048
N_PAD = 10240


def _rowsum_body(x_ref, r_ref):
    r_ref[...] = jnp.sum(x_ref[...].T, axis=0)


def _rowsum(x):
    grid = N_PAD // _ROWS_PER_BLOCK
    return pl.pallas_call(
        _rowsum_body,
        grid=(grid,),
        in_specs=[pl.BlockSpec((_ROWS_PER_BLOCK, D_FEAT), lambda i: (i, 0))],
        out_specs=pl.BlockSpec((_ROWS_PER_BLOCK,), lambda i: (i,)),
        out_shape=jax.ShapeDtypeStruct((N_PAD,), jnp.float32),
    )(x)


def _sc_scatter(r, ei4, zeros):
    mesh = plsc.VectorSubcoreMesh(core_axis_name="c", subcore_axis_name="s")

    @functools.partial(
        pl.kernel,
        out_type=jax.ShapeDtypeStruct((N_CORES, N_NODES), jnp.float32),
        mesh=mesh,
        scratch_types=[
            pltpu.VMEM((NCHUNK, CHUNK), jnp.int32),
            pltpu.VMEM((NCHUNK, CHUNK), jnp.int32),
            pltpu.VMEM((NCHUNK, CHUNK), jnp.float32),
            pltpu.VMEM_SHARED((N_PAD,), jnp.float32),
            pltpu.VMEM_SHARED((N_NODES,), jnp.float32),
            pltpu.SemaphoreType.DMA,
            pltpu.SemaphoreType.DMA,
        ],
    )
    def scatter_kernel(r_hbm, ei_hbm, z_hbm, out_hbm,
                       src_v, dst_v, val_v, r_sh, acc_sh, gsem, ssem):
        cid = lax.axis_index("c")
        sid = lax.axis_index("s")
        wid = sid * N_CORES + cid

        @pl.when(sid == 0)
        def _():
            pltpu.sync_copy(r_hbm, r_sh)
            pltpu.sync_copy(z_hbm, acc_sh)

        pltpu.sync_copy(ei_hbm.at[0, wid], src_v)
        pltpu.sync_copy(ei_hbm.at[1, wid], dst_v)
        plsc.subcore_barrier()

        def fire_gather(j, carry):
            pltpu.async_copy(r_sh.at[src_v.at[j]], val_v.at[j], gsem)
            return carry

        lax.fori_loop(0, NCHUNK, fire_gather, 0)

        def drain_gather(j, carry):
            pltpu.make_async_copy(r_sh.at[src_v.at[j]], val_v.at[j],
                                  gsem).wait()
            return carry

        lax.fori_loop(0, NCHUNK, drain_gather, 0)

        def fire_scatter(j, carry):
            pltpu.async_copy(val_v.at[j], acc_sh.at[dst_v.at[j]], ssem,
                             add=True)
            return carry

        lax.fori_loop(0, NCHUNK, fire_scatter, 0)

        def drain_scatter(j, carry):
            pltpu.make_async_copy(val_v.at[j], acc_sh.at[dst_v.at[j]],
                                  ssem).wait()
            return carry

        lax.fori_loop(0, NCHUNK, drain_scatter, 0)
        plsc.subcore_barrier()

        @pl.when(sid == 0)
        def _():
            pltpu.sync_copy(acc_sh, out_hbm.at[cid])

    return scatter_kernel(r, ei4, zeros)


def _final_body(p_ref, o_ref):
    s = p_ref[0, :] + p_ref[1, :]
    t = jnp.where(s > 0, s, 0.1 * s)
    h = 16.0 * t
    u = jnp.where(h > 0, h, 0.1 * h)
    o_ref[...] = (10.0 * u).astype(jnp.int32)


def _finalize(parts):
    return pl.pallas_call(
        _final_body,
        out_shape=jax.ShapeDtypeStruct((N_NODES,), jnp.int32),
    )(parts)


def kernel(x, edge_index):
    ei4 = edge_index.astype(jnp.int32).reshape(2, NW, NCHUNK, CHUNK)
    zeros = jnp.zeros((N_NODES,), jnp.float32)
    r = _rowsum(x)
    parts = _sc_scatter(r, ei4, zeros)
    return _finalize(parts)
